# baseline (speedup 1.0000x reference)
import dataclasses
import functools

import jax
import jax.numpy as jnp
from jax import lax
from jax.experimental import pallas as pl
from jax.experimental.pallas import tpu as pltpu
from jax.experimental.pallas import tpu_sc as plsc

E = 320000
N_PTS = 10000
N_VIEWS = 500
NVP = 512
D = 128
DP = D // 2
L = 16

NC = 2
NS = 16
NW = NC * NS
BPW = E // NW
C = 400

BE = 2560
NB = E // BE


def _sc_gather_pt_packed(pt_tbl, pt_idx):
    mesh = plsc.VectorSubcoreMesh(core_axis_name="c", subcore_axis_name="s")
    cp = pltpu.CompilerParams()
    if "needs_layout_passes" in pltpu.CompilerParams.__dataclass_fields__:
        cp = dataclasses.replace(cp, needs_layout_passes=False)

    @functools.partial(
        pl.kernel,
        mesh=mesh,
        compiler_params=cp,
        out_type=jax.ShapeDtypeStruct((E, DP), jnp.int32),
        scratch_types=[
            pltpu.VMEM((C,), jnp.int32),
            pltpu.VMEM((C, D), jnp.float32),
            pltpu.VMEM((C, DP), jnp.int32),
            pltpu.SemaphoreType.DMA,
        ],
    )
    def k(pt_hbm, pi_hbm, o_hbm, pi_v, rp_v, po_v, sem):
        wid = lax.axis_index("s") * NC + lax.axis_index("c")
        base = wid * BPW

        @pl.loop(0, BPW, step=C)
        def _(off):
            s = base + off
            pltpu.sync_copy(pi_hbm.at[pl.ds(s, C)], pi_v)
            pltpu.async_copy(pt_hbm.at[pi_v], rp_v, sem).wait()

            @pl.loop(0, C)
            def _(i):
                for g in range(DP // L):
                    a = rp_v[i, pl.ds(L * g, L)]
                    b2 = rp_v[i, pl.ds(DP + L * g, L)]
                    ai = plsc.bitcast(a, jnp.int32)
                    bi = plsc.bitcast(b2, jnp.int32)
                    lo = lax.shift_right_logical(ai + 32768, 16)
                    hi = (bi + 32768) & jnp.int32(-65536)
                    po_v[i, pl.ds(L * g, L)] = lo | hi

            pltpu.sync_copy(po_v, o_hbm.at[pl.ds(s, C)])

    return k(pt_tbl, pt_idx)


def _tc_body(v_ref, p_ref, ci_ref, iot_ref, w_ref, vw_ref, bg_ref, o_ref):
    vb = v_ref[...].astype(jnp.bfloat16)
    wb = w_ref[...].astype(jnp.bfloat16)
    acc = lax.dot_general(
        vb, wb, (((1,), (1,)), ((), ())),
        preferred_element_type=jnp.float32,
    )
    cam = ci_ref[0, 0, :].astype(jnp.int16)
    oh = jnp.where(cam[:, None] == iot_ref[...],
                   jnp.bfloat16(1), jnp.bfloat16(0))
    view = lax.dot_general(
        oh, vw_ref[...], (((1,), (0,)), ((), ())),
        preferred_element_type=jnp.float32,
    )
    xi = p_ref[...]
    lo = lax.bitcast_convert_type(lax.shift_left(xi, 16), jnp.float32)
    hi = lax.bitcast_convert_type(xi & jnp.int32(-65536), jnp.float32)
    ptf = jnp.concatenate([lo, hi], axis=1)
    o_ref[...] = (acc + view + ptf + bg_ref[...]) * 0.25


def kernel(values, scenepoint_features, view_features, global_features,
           cam_idx, pt_idx, W, b):
    pt_rows = _sc_gather_pt_packed(scenepoint_features,
                                   pt_idx.astype(jnp.int32))

    ci3 = cam_idx.astype(jnp.int32).reshape(NB, 1, BE)
    iot = lax.iota(jnp.int16, NVP)[None, :]
    vw_pad = jnp.zeros((NVP, D), jnp.bfloat16).at[:N_VIEWS].set(
        view_features.astype(jnp.bfloat16))
    bg = (b + global_features)[None, :]

    out = pl.pallas_call(
        _tc_body,
        grid=(NB,),
        in_specs=[
            pl.BlockSpec((BE, D), lambda i: (i, 0)),
            pl.BlockSpec((BE, DP), lambda i: (i, 0)),
            pl.BlockSpec((1, 1, BE), lambda i: (i, 0, 0)),
            pl.BlockSpec((1, NVP), lambda i: (0, 0)),
            pl.BlockSpec((D, D), lambda i: (0, 0)),
            pl.BlockSpec((NVP, D), lambda i: (0, 0)),
            pl.BlockSpec((1, D), lambda i: (0, 0)),
        ],
        out_specs=pl.BlockSpec((BE, D), lambda i: (i, 0)),
        out_shape=jax.ShapeDtypeStruct((E, D), jnp.float32),
    )(values, pt_rows, ci3, iot, W, vw_pad, bg)
    return out


# speedup vs baseline: 1.2206x; 1.2206x over previous
import functools

import jax
import jax.numpy as jnp
from jax import lax
from jax.experimental import pallas as pl
from jax.experimental.pallas import tpu as pltpu
from jax.experimental.pallas import tpu_sc as plsc

E = 320000
N_PTS = 10000
N_VIEWS = 500
NVP = 512
D = 128

NC = 2
NS = 16
NW = NC * NS
BPW = E // NW
C = 400

BE = 4000
NB = E // BE


def _sc_gather_pt(pt_tbl, pt_idx):
    mesh = plsc.VectorSubcoreMesh(core_axis_name="c", subcore_axis_name="s")

    @functools.partial(
        pl.kernel,
        mesh=mesh,
        out_type=jax.ShapeDtypeStruct((E, D), jnp.float32),
        scratch_types=[
            pltpu.VMEM((C,), jnp.int32),
            pltpu.VMEM((C, D), jnp.float32),
            pltpu.SemaphoreType.DMA,
        ],
    )
    def k(pt_hbm, pi_hbm, o_hbm, pi_v, rp_v, sem):
        wid = lax.axis_index("s") * NC + lax.axis_index("c")
        base = wid * BPW

        @pl.loop(0, BPW, step=C)
        def _(off):
            s = base + off
            pltpu.sync_copy(pi_hbm.at[pl.ds(s, C)], pi_v)
            pltpu.async_copy(pt_hbm.at[pi_v], rp_v, sem).wait()
            pltpu.sync_copy(rp_v, o_hbm.at[pl.ds(s, C)])

    return k(pt_tbl, pt_idx)


def _tc_body(v_ref, p_ref, ci_ref, iot_ref, w_ref, vw_ref, bg_ref, o_ref):
    vb = v_ref[...].astype(jnp.bfloat16)
    wb = w_ref[...].astype(jnp.bfloat16)
    acc = lax.dot_general(
        vb, wb, (((1,), (1,)), ((), ())),
        preferred_element_type=jnp.float32,
    )
    cam = ci_ref[0, 0, :].astype(jnp.int16)
    oh = jnp.where(cam[:, None] == iot_ref[...],
                   jnp.bfloat16(1), jnp.bfloat16(0))
    view = lax.dot_general(
        oh, vw_ref[...], (((1,), (0,)), ((), ())),
        preferred_element_type=jnp.float32,
    )
    o_ref[...] = (acc + view + p_ref[...] + bg_ref[...]) * 0.25


def kernel(values, scenepoint_features, view_features, global_features,
           cam_idx, pt_idx, W, b):
    pt_rows = _sc_gather_pt(scenepoint_features, pt_idx.astype(jnp.int32))

    ci3 = cam_idx.astype(jnp.int32).reshape(NB, 1, BE)
    iot = lax.iota(jnp.int16, NVP)[None, :]
    vw_pad = jnp.zeros((NVP, D), jnp.bfloat16).at[:N_VIEWS].set(
        view_features.astype(jnp.bfloat16))
    bg = (b + global_features)[None, :]

    out = pl.pallas_call(
        _tc_body,
        grid=(NB,),
        in_specs=[
            pl.BlockSpec((BE, D), lambda i: (i, 0)),
            pl.BlockSpec((BE, D), lambda i: (i, 0)),
            pl.BlockSpec((1, 1, BE), lambda i: (i, 0, 0)),
            pl.BlockSpec((1, NVP), lambda i: (0, 0)),
            pl.BlockSpec((D, D), lambda i: (0, 0)),
            pl.BlockSpec((NVP, D), lambda i: (0, 0)),
            pl.BlockSpec((1, D), lambda i: (0, 0)),
        ],
        out_specs=pl.BlockSpec((BE, D), lambda i: (i, 0)),
        out_shape=jax.ShapeDtypeStruct((E, D), jnp.float32),
    )(values, pt_rows, ci3, iot, W, vw_pad, bg)
    return out


# speedup vs baseline: 1.3115x; 1.0744x over previous
import functools

import jax
import jax.numpy as jnp
from jax import lax
from jax.experimental import pallas as pl
from jax.experimental.pallas import tpu as pltpu
from jax.experimental.pallas import tpu_sc as plsc

E = 320000
N_PTS = 10000
N_VIEWS = 500
NVP = 512
D = 128

NC = 2
NS = 16
NW = NC * NS
BPW = E // NW
C = 400

BE = 8000
NB = E // BE


def _sc_gather_pt(pt_tbl, pt_idx):
    mesh = plsc.VectorSubcoreMesh(core_axis_name="c", subcore_axis_name="s")

    @functools.partial(
        pl.kernel,
        mesh=mesh,
        out_type=jax.ShapeDtypeStruct((E, D), jnp.float32),
        scratch_types=[
            pltpu.VMEM((C,), jnp.int32),
            pltpu.VMEM((C, D), jnp.float32),
            pltpu.SemaphoreType.DMA,
        ],
    )
    def k(pt_hbm, pi_hbm, o_hbm, pi_v, rp_v, sem):
        wid = lax.axis_index("s") * NC + lax.axis_index("c")
        base = wid * BPW

        @pl.loop(0, BPW, step=C)
        def _(off):
            s = base + off
            pltpu.sync_copy(pi_hbm.at[pl.ds(s, C)], pi_v)
            pltpu.async_copy(pt_hbm.at[pi_v], rp_v, sem).wait()
            pltpu.sync_copy(rp_v, o_hbm.at[pl.ds(s, C)])

    return k(pt_tbl, pt_idx)


def _tc_body(v_ref, p_ref, ci_ref, iot_ref, w_ref, vw_ref, bg_ref, o_ref):
    vb = v_ref[...].astype(jnp.bfloat16)
    wb = w_ref[...].astype(jnp.bfloat16)
    acc = lax.dot_general(
        vb, wb, (((1,), (1,)), ((), ())),
        preferred_element_type=jnp.float32,
    )
    cam = ci_ref[0, 0, :].astype(jnp.int16)
    oh = jnp.where(cam[:, None] == iot_ref[...],
                   jnp.bfloat16(1), jnp.bfloat16(0))
    view = lax.dot_general(
        oh, vw_ref[...], (((1,), (0,)), ((), ())),
        preferred_element_type=jnp.float32,
    )
    o_ref[...] = (acc + view + p_ref[...] + bg_ref[...]) * 0.25


def kernel(values, scenepoint_features, view_features, global_features,
           cam_idx, pt_idx, W, b):
    pt_rows = _sc_gather_pt(scenepoint_features, pt_idx.astype(jnp.int32))

    ci3 = cam_idx.astype(jnp.int32).reshape(NB, 1, BE)
    iot = lax.iota(jnp.int16, NVP)[None, :]
    vw_pad = jnp.zeros((NVP, D), jnp.bfloat16).at[:N_VIEWS].set(
        view_features.astype(jnp.bfloat16))
    bg = (b + global_features)[None, :]

    out = pl.pallas_call(
        _tc_body,
        grid=(NB,),
        in_specs=[
            pl.BlockSpec((BE, D), lambda i: (i, 0)),
            pl.BlockSpec((BE, D), lambda i: (i, 0)),
            pl.BlockSpec((1, 1, BE), lambda i: (i, 0, 0)),
            pl.BlockSpec((1, NVP), lambda i: (0, 0)),
            pl.BlockSpec((D, D), lambda i: (0, 0)),
            pl.BlockSpec((NVP, D), lambda i: (0, 0)),
            pl.BlockSpec((1, D), lambda i: (0, 0)),
        ],
        out_specs=pl.BlockSpec((BE, D), lambda i: (i, 0)),
        out_shape=jax.ShapeDtypeStruct((E, D), jnp.float32),
    )(values, pt_rows, ci3, iot, W, vw_pad, bg)
    return out
